# baseline (speedup 1.0000x reference)
import functools

import jax
import jax.numpy as jnp
from jax import lax
from jax.experimental import pallas as pl
from jax.experimental.pallas import tpu as pltpu
from jax.experimental.pallas import tpu_sc as plsc

N_NODES_ = 10000
N_EDGES_ = 320000
DIN = 128
DH = 128
DOUT = 40

NC = 2
NS = 16
NW = NC * NS
CHUNK = 128
NPAD = 10240
DUMMY = N_NODES_
NCHUNKS = (N_EDGES_ + CHUNK - 1) // CHUNK
CPT = ((NCHUNKS + NW - 1) // NW + 7) // 8 * 8
NCHUNKS_PAD = CPT * NW
EPAD = NCHUNKS_PAD * CHUNK
ZR = NPAD // NS

_MESH = plsc.VectorSubcoreMesh(core_axis_name="c", subcore_axis_name="s")


DW = 8


def _sc_degree(e2, cvals, zeros2):
    @functools.partial(
        pl.kernel,
        out_type=jax.ShapeDtypeStruct((NC, NPAD, DW), jnp.float32),
        mesh=_MESH,
        compiler_params=pltpu.CompilerParams(use_tc_tiling_on_sc=False),
        scratch_types=[
            pltpu.VMEM((CPT, CHUNK), jnp.int32),
            pltpu.VMEM((CPT, CHUNK), jnp.int32),
            pltpu.VMEM((CHUNK, DW), jnp.float32),
            pltpu.VMEM((CHUNK, DW), jnp.float32),
            pltpu.VMEM_SHARED((NPAD, DW), jnp.float32),
            pltpu.SemaphoreType.DMA,
        ],
    )
    def deg_kernel(e2_h, cvals_h, zeros_h, out_h, src_v, dst_v, va, vb,
                   acc_sh, sem_s):
        c = lax.axis_index("c")
        s = lax.axis_index("s")
        wid = s * NC + c
        pltpu.sync_copy(e2_h.at[0, pl.ds(wid * CPT, CPT)], src_v)
        pltpu.sync_copy(e2_h.at[1, pl.ds(wid * CPT, CPT)], dst_v)
        pltpu.sync_copy(cvals_h.at[0], va)
        pltpu.sync_copy(cvals_h.at[1], vb)
        pltpu.sync_copy(zeros_h, acc_sh.at[pl.ds(s * ZR, ZR)])
        plsc.subcore_barrier()

        def step(j, carry):
            pltpu.async_copy(va, acc_sh.at[src_v.at[j]], sem_s, add=True)
            pltpu.async_copy(vb, acc_sh.at[dst_v.at[j]], sem_s, add=True)
            return carry

        lax.fori_loop(0, CPT, step, 0, unroll=False)

        def drain(j, carry):
            pltpu.make_async_copy(va, acc_sh.at[src_v.at[0]], sem_s).wait()
            pltpu.make_async_copy(vb, acc_sh.at[dst_v.at[0]], sem_s).wait()
            return carry

        lax.fori_loop(0, CPT, drain, 0, unroll=False)
        plsc.subcore_barrier()
        pltpu.sync_copy(acc_sh.at[pl.ds(s * ZR, ZR)],
                        out_h.at[c, pl.ds(s * ZR, ZR)])

    return deg_kernel(e2, cvals, zeros2)


BLK_I = 8
NBLK = CPT // BLK_I


def _make_sc_scatter(d_feat, stage=False):
    scratch = [
        pltpu.VMEM((2, BLK_I, CHUNK), jnp.int32),
        pltpu.VMEM((CPT, CHUNK), jnp.int32),
        pltpu.VMEM((2, CHUNK, d_feat), jnp.float32),
        pltpu.VMEM_SHARED((NPAD, d_feat), jnp.float32),
        pltpu.SemaphoreType.DMA,
        pltpu.SemaphoreType.DMA,
    ]
    if stage:
        scratch.append(pltpu.VMEM_SHARED((NPAD, d_feat), jnp.float32))

    @functools.partial(
        pl.kernel,
        out_type=jax.ShapeDtypeStruct((NC, NPAD, d_feat), jnp.float32),
        mesh=_MESH,
        compiler_params=pltpu.CompilerParams(use_tc_tiling_on_sc=False),
        scratch_types=scratch,
    )
    def scat_kernel(y_h, e2_h, zeros_h, out_h, srcb_v, dst_v, rows_v, acc_sh,
                    sem_g, sem_i, *maybe_ysh):
        c = lax.axis_index("c")
        s = lax.axis_index("s")
        wid = s * NC + c
        base = wid * CPT
        pltpu.sync_copy(e2_h.at[1, pl.ds(base, CPT)], dst_v)
        pltpu.sync_copy(e2_h.at[0, pl.ds(base, BLK_I)], srcb_v.at[0])
        pltpu.async_copy(e2_h.at[0, pl.ds(base + BLK_I, BLK_I)],
                         srcb_v.at[1], sem_i)
        pltpu.sync_copy(zeros_h, acc_sh.at[pl.ds(s * ZR, ZR)])
        if stage:
            y_t = maybe_ysh[0]
            pltpu.sync_copy(y_h.at[pl.ds(s * ZR, ZR)],
                            y_t.at[pl.ds(s * ZR, ZR)])
        else:
            y_t = y_h
        plsc.subcore_barrier()
        pltpu.async_copy(y_t.at[srcb_v.at[0, 0]], rows_v.at[0], sem_g)

        def outer(g, carry):
            gm = g % 2
            gn = (g + 1) % 2
            for k in range(BLK_I):
                j = g * BLK_I + k
                slot = rows_v.at[k % 2]
                nslot = rows_v.at[(k + 1) % 2]
                if k < BLK_I - 1:
                    pltpu.async_copy(y_t.at[srcb_v.at[gm, k + 1]], nslot,
                                     sem_g)
                else:
                    @pl.when(g + 1 < NBLK)
                    def _():
                        pltpu.make_async_copy(
                            e2_h.at[0, pl.ds(base, BLK_I)],
                            srcb_v.at[gn], sem_i).wait()
                        pltpu.async_copy(y_t.at[srcb_v.at[gn, 0]], nslot,
                                         sem_g)

                    @pl.when(g + 2 < NBLK)
                    def _():
                        pltpu.async_copy(
                            e2_h.at[0, pl.ds(base + (g + 2) * BLK_I, BLK_I)],
                            srcb_v.at[gm], sem_i)

                pltpu.make_async_copy(y_t.at[srcb_v.at[gm, k]], slot,
                                      sem_g).wait()
                pltpu.sync_copy(slot, acc_sh.at[dst_v.at[j]], add=True)
            return carry

        lax.fori_loop(0, NBLK, outer, 0, unroll=False)
        plsc.subcore_barrier()
        pltpu.sync_copy(acc_sh.at[pl.ds(s * ZR, ZR)],
                        out_h.at[c, pl.ds(s * ZR, ZR)])

    return scat_kernel


DHH = DH // 2
_sc_scatter_40 = _make_sc_scatter(DOUT, stage=True)

CPT2 = NCHUNKS_PAD // NS
NBLK2 = CPT2 // BLK_I


def _sc_scatter_split(y2h, e2, zeros64):
    @functools.partial(
        pl.kernel,
        out_type=jax.ShapeDtypeStruct((NC, NPAD, DHH), jnp.float32),
        mesh=_MESH,
        compiler_params=pltpu.CompilerParams(use_tc_tiling_on_sc=False),
        scratch_types=[
            pltpu.VMEM((2, BLK_I, CHUNK), jnp.int32),
            pltpu.VMEM((CPT2, CHUNK), jnp.int32),
            pltpu.VMEM((2, CHUNK, DHH), jnp.float32),
            pltpu.VMEM_SHARED((NPAD, DHH), jnp.float32),
            pltpu.VMEM_SHARED((NPAD, DHH), jnp.float32),
            pltpu.SemaphoreType.DMA,
            pltpu.SemaphoreType.DMA,
        ],
    )
    def scat_kernel(y_h, e2_h, zeros_h, out_h, srcb_v, dst_v, rows_v, acc_sh,
                    y_t, sem_g, sem_i):
        c = lax.axis_index("c")
        s = lax.axis_index("s")
        base = s * CPT2
        pltpu.sync_copy(e2_h.at[1, pl.ds(base, CPT2)], dst_v)
        pltpu.sync_copy(e2_h.at[0, pl.ds(base, BLK_I)], srcb_v.at[0])
        pltpu.async_copy(e2_h.at[0, pl.ds(base + BLK_I, BLK_I)],
                         srcb_v.at[1], sem_i)
        pltpu.sync_copy(zeros_h, acc_sh.at[pl.ds(s * ZR, ZR)])
        pltpu.sync_copy(y_h.at[c, pl.ds(s * ZR, ZR)],
                        y_t.at[pl.ds(s * ZR, ZR)])
        plsc.subcore_barrier()
        pltpu.async_copy(y_t.at[srcb_v.at[0, 0]], rows_v.at[0], sem_g)

        def outer(g, carry):
            gm = g % 2
            gn = (g + 1) % 2
            for k in range(BLK_I):
                j = g * BLK_I + k
                slot = rows_v.at[k % 2]
                nslot = rows_v.at[(k + 1) % 2]
                if k < BLK_I - 1:
                    pltpu.async_copy(y_t.at[srcb_v.at[gm, k + 1]], nslot,
                                     sem_g)
                else:
                    @pl.when(g + 1 < NBLK2)
                    def _():
                        pltpu.make_async_copy(
                            e2_h.at[0, pl.ds(base, BLK_I)],
                            srcb_v.at[gn], sem_i).wait()
                        pltpu.async_copy(y_t.at[srcb_v.at[gn, 0]], nslot,
                                         sem_g)

                    @pl.when(g + 2 < NBLK2)
                    def _():
                        pltpu.async_copy(
                            e2_h.at[0, pl.ds(base + (g + 2) * BLK_I, BLK_I)],
                            srcb_v.at[gm], sem_i)

                pltpu.make_async_copy(y_t.at[srcb_v.at[gm, k]], slot,
                                      sem_g).wait()
                pltpu.sync_copy(slot, acc_sh.at[dst_v.at[j]], add=True)
            return carry

        lax.fori_loop(0, NBLK2, outer, 0, unroll=False)
        plsc.subcore_barrier()
        pltpu.sync_copy(acc_sh.at[pl.ds(s * ZR, ZR)],
                        out_h.at[c, pl.ds(s * ZR, ZR)])

    return scat_kernel(y2h, e2, zeros64)


def _norms(dref):
    d = dref[0] + dref[1]
    n_src = lax.rsqrt(jnp.maximum(d[:, 0:1], 1.0))
    n_dst = lax.rsqrt(jnp.maximum(d[:, 1:2], 1.0))
    return n_src, n_dst


_BLK = 1024


def _tc_prep(x_pad, W1, deg_p):
    def body(x_ref, w_ref, d_ref, o_ref):
        n_src, _ = _norms(d_ref)
        y = jnp.dot(x_ref[...], w_ref[...],
                    preferred_element_type=jnp.float32) * n_src
        o_ref[0] = y[:, :DHH]
        o_ref[1] = y[:, DHH:]

    return pl.pallas_call(
        body,
        grid=(NPAD // _BLK,),
        in_specs=[
            pl.BlockSpec((_BLK, DIN), lambda i: (i, 0)),
            pl.BlockSpec((DIN, DH), lambda i: (0, 0)),
            pl.BlockSpec((NC, _BLK, DW), lambda i: (0, i, 0)),
        ],
        out_specs=pl.BlockSpec((NC, _BLK, DHH), lambda i: (0, i, 0)),
        out_shape=jax.ShapeDtypeStruct((NC, NPAD, DHH), jnp.float32),
    )(x_pad, W1, deg_p)


def _tc_mid(agg1, deg_p, W2, b1r):
    def body(a_ref, d_ref, w_ref, bias_ref, o_ref):
        n_src, n_dst = _norms(d_ref)
        h = jnp.concatenate([a_ref[0], a_ref[1]],
                            axis=1) * n_dst + bias_ref[...]
        h = jnp.maximum(h, 0.0)
        o_ref[...] = jnp.dot(h, w_ref[...],
                             preferred_element_type=jnp.float32) * n_src

    return pl.pallas_call(
        body,
        grid=(NPAD // _BLK,),
        in_specs=[
            pl.BlockSpec((NC, _BLK, DHH), lambda i: (0, i, 0)),
            pl.BlockSpec((NC, _BLK, DW), lambda i: (0, i, 0)),
            pl.BlockSpec((DH, DOUT), lambda i: (0, 0)),
            pl.BlockSpec((1, DH), lambda i: (0, 0)),
        ],
        out_specs=pl.BlockSpec((_BLK, DOUT), lambda i: (i, 0)),
        out_shape=jax.ShapeDtypeStruct((NPAD, DOUT), jnp.float32),
    )(agg1, deg_p, W2, b1r)


def _tc_final(agg2, deg_p, b2r):
    def body(a_ref, d_ref, b_ref, o_ref):
        _, n_dst = _norms(d_ref)
        o_ref[...] = (a_ref[0] + a_ref[1]) * n_dst + b_ref[...]

    return pl.pallas_call(
        body,
        grid=(NPAD // _BLK,),
        in_specs=[
            pl.BlockSpec((NC, _BLK, DOUT), lambda i: (0, i, 0)),
            pl.BlockSpec((NC, _BLK, DW), lambda i: (0, i, 0)),
            pl.BlockSpec((1, DOUT), lambda i: (0, 0)),
        ],
        out_specs=pl.BlockSpec((_BLK, DOUT), lambda i: (i, 0)),
        out_shape=jax.ShapeDtypeStruct((NPAD, DOUT), jnp.float32),
    )(agg2, deg_p, b2r)


def kernel(in_feat, edge_index, W1, b1, W2, b2):
    pad_e = EPAD - N_EDGES_
    epad = jnp.full((2, pad_e), DUMMY, dtype=jnp.int32)
    e2 = jnp.concatenate([edge_index, epad], axis=1).reshape(2, NCHUNKS_PAD, CHUNK)
    x_pad = jnp.zeros((NPAD, DIN), jnp.float32).at[:N_NODES_].set(in_feat)
    b1r = b1.reshape(1, DH)
    b2r = b2.reshape(1, DOUT)
    eye2 = jnp.zeros((2, DW), jnp.float32).at[0, 0].set(1.0).at[1, 1].set(1.0)
    cvals = jnp.tile(eye2[:, None, :], (1, CHUNK, 1))
    zeros2 = jnp.zeros((ZR, DW), jnp.float32)
    zeros64 = jnp.zeros((ZR, DHH), jnp.float32)
    zeros40 = jnp.zeros((ZR, DOUT), jnp.float32)

    deg_p = _sc_degree(e2, cvals, zeros2)
    y1 = _tc_prep(x_pad, W1, deg_p)
    agg1 = _sc_scatter_split(y1, e2, zeros64)
    y2 = _tc_mid(agg1, deg_p, W2, b1r)
    agg2 = _sc_scatter_40(y2, e2, zeros40)
    out = _tc_final(agg2, deg_p, b2r)
    return out[:N_NODES_]


# speedup vs baseline: 1.2369x; 1.2369x over previous
import functools

import jax
import jax.numpy as jnp
from jax import lax
from jax.experimental import pallas as pl
from jax.experimental.pallas import tpu as pltpu
from jax.experimental.pallas import tpu_sc as plsc

N_NODES_ = 10000
N_EDGES_ = 320000
DIN = 128
DH = 128
DOUT = 40

NC = 2
NS = 16
NW = NC * NS
CHUNK = 128
NPAD = 10240
DUMMY = N_NODES_
NCHUNKS = (N_EDGES_ + CHUNK - 1) // CHUNK
CPT = ((NCHUNKS + NW - 1) // NW + 7) // 8 * 8
NCHUNKS_PAD = CPT * NW
EPAD = NCHUNKS_PAD * CHUNK
ZR = NPAD // NS

_MESH = plsc.VectorSubcoreMesh(core_axis_name="c", subcore_axis_name="s")


DW = 8


def _sc_degree(e2, cvals, zeros2):
    @functools.partial(
        pl.kernel,
        out_type=jax.ShapeDtypeStruct((NC, NPAD, DW), jnp.float32),
        mesh=_MESH,
        compiler_params=pltpu.CompilerParams(use_tc_tiling_on_sc=False),
        scratch_types=[
            pltpu.VMEM((CPT, CHUNK), jnp.int32),
            pltpu.VMEM((CPT, CHUNK), jnp.int32),
            pltpu.VMEM((CHUNK, DW), jnp.float32),
            pltpu.VMEM((CHUNK, DW), jnp.float32),
            pltpu.VMEM_SHARED((NPAD, DW), jnp.float32),
            pltpu.SemaphoreType.DMA,
        ],
    )
    def deg_kernel(e2_h, cvals_h, zeros_h, out_h, src_v, dst_v, va, vb,
                   acc_sh, sem_s):
        c = lax.axis_index("c")
        s = lax.axis_index("s")
        wid = s * NC + c
        pltpu.sync_copy(e2_h.at[0, pl.ds(wid * CPT, CPT)], src_v)
        pltpu.sync_copy(e2_h.at[1, pl.ds(wid * CPT, CPT)], dst_v)
        pltpu.sync_copy(cvals_h.at[0], va)
        pltpu.sync_copy(cvals_h.at[1], vb)
        pltpu.sync_copy(zeros_h, acc_sh.at[pl.ds(s * ZR, ZR)])
        plsc.subcore_barrier()

        def step(j, carry):
            pltpu.async_copy(va, acc_sh.at[src_v.at[j]], sem_s, add=True)
            pltpu.async_copy(vb, acc_sh.at[dst_v.at[j]], sem_s, add=True)
            return carry

        lax.fori_loop(0, CPT, step, 0, unroll=False)

        def drain(j, carry):
            pltpu.make_async_copy(va, acc_sh.at[src_v.at[0]], sem_s).wait()
            pltpu.make_async_copy(vb, acc_sh.at[dst_v.at[0]], sem_s).wait()
            return carry

        lax.fori_loop(0, CPT, drain, 0, unroll=False)
        plsc.subcore_barrier()
        pltpu.sync_copy(acc_sh.at[pl.ds(s * ZR, ZR)],
                        out_h.at[c, pl.ds(s * ZR, ZR)])

    return deg_kernel(e2, cvals, zeros2)


BLK_I = 8
NBLK = CPT // BLK_I


def _make_sc_scatter(d_feat, stage=False):
    scratch = [
        pltpu.VMEM((2, BLK_I, CHUNK), jnp.int32),
        pltpu.VMEM((CPT, CHUNK), jnp.int32),
        pltpu.VMEM((2, CHUNK, d_feat), jnp.float32),
        pltpu.VMEM_SHARED((NPAD, d_feat), jnp.float32),
        pltpu.SemaphoreType.DMA,
        pltpu.SemaphoreType.DMA,
    ]
    if stage:
        scratch.append(pltpu.VMEM_SHARED((NPAD, d_feat), jnp.float32))

    @functools.partial(
        pl.kernel,
        out_type=jax.ShapeDtypeStruct((NC, NPAD, d_feat), jnp.float32),
        mesh=_MESH,
        compiler_params=pltpu.CompilerParams(use_tc_tiling_on_sc=False),
        scratch_types=scratch,
    )
    def scat_kernel(y_h, e2_h, zeros_h, out_h, srcb_v, dst_v, rows_v, acc_sh,
                    sem_g, sem_i, *maybe_ysh):
        c = lax.axis_index("c")
        s = lax.axis_index("s")
        wid = s * NC + c
        base = wid * CPT
        pltpu.sync_copy(e2_h.at[1, pl.ds(base, CPT)], dst_v)
        pltpu.sync_copy(e2_h.at[0, pl.ds(base, BLK_I)], srcb_v.at[0])
        pltpu.async_copy(e2_h.at[0, pl.ds(base + BLK_I, BLK_I)],
                         srcb_v.at[1], sem_i)
        pltpu.sync_copy(zeros_h, acc_sh.at[pl.ds(s * ZR, ZR)])
        if stage:
            y_t = maybe_ysh[0]
            pltpu.sync_copy(y_h.at[pl.ds(s * ZR, ZR)],
                            y_t.at[pl.ds(s * ZR, ZR)])
        else:
            y_t = y_h
        plsc.subcore_barrier()
        pltpu.async_copy(y_t.at[srcb_v.at[0, 0]], rows_v.at[0], sem_g)

        def outer(g, carry):
            gm = g % 2
            gn = (g + 1) % 2
            for k in range(BLK_I):
                j = g * BLK_I + k
                slot = rows_v.at[k % 2]
                nslot = rows_v.at[(k + 1) % 2]
                if k < BLK_I - 1:
                    pltpu.async_copy(y_t.at[srcb_v.at[gm, k + 1]], nslot,
                                     sem_g)
                else:
                    @pl.when(g + 1 < NBLK)
                    def _():
                        pltpu.make_async_copy(
                            e2_h.at[0, pl.ds(base, BLK_I)],
                            srcb_v.at[gn], sem_i).wait()
                        pltpu.async_copy(y_t.at[srcb_v.at[gn, 0]], nslot,
                                         sem_g)

                    @pl.when(g + 2 < NBLK)
                    def _():
                        pltpu.async_copy(
                            e2_h.at[0, pl.ds(base + (g + 2) * BLK_I, BLK_I)],
                            srcb_v.at[gm], sem_i)

                pltpu.make_async_copy(y_t.at[srcb_v.at[gm, k]], slot,
                                      sem_g).wait()
                pltpu.sync_copy(slot, acc_sh.at[dst_v.at[j]], add=True)
            return carry

        lax.fori_loop(0, NBLK, outer, 0, unroll=False)
        plsc.subcore_barrier()
        pltpu.sync_copy(acc_sh.at[pl.ds(s * ZR, ZR)],
                        out_h.at[c, pl.ds(s * ZR, ZR)])

    return scat_kernel


DHH = DH // 2
_sc_scatter_40 = _make_sc_scatter(DOUT, stage=True)

CPT2 = NCHUNKS_PAD // NS
NBLK2 = CPT2 // BLK_I


def _sc_scatter_split(y2h, e2, zeros64):
    @functools.partial(
        pl.kernel,
        out_type=jax.ShapeDtypeStruct((NC, NPAD, DHH), jnp.bfloat16),
        mesh=_MESH,
        compiler_params=pltpu.CompilerParams(use_tc_tiling_on_sc=False),
        scratch_types=[
            pltpu.VMEM((2, BLK_I, CHUNK), jnp.int32),
            pltpu.VMEM((CPT2, CHUNK), jnp.int32),
            pltpu.VMEM((2, CHUNK, DHH), jnp.bfloat16),
            pltpu.VMEM_SHARED((NPAD, DHH), jnp.bfloat16),
            pltpu.VMEM_SHARED((NPAD, DHH), jnp.bfloat16),
            pltpu.SemaphoreType.DMA,
            pltpu.SemaphoreType.DMA,
        ],
    )
    def scat_kernel(y_h, e2_h, zeros_h, out_h, srcb_v, dst_v, rows_v, acc_sh,
                    y_t, sem_g, sem_i):
        c = lax.axis_index("c")
        s = lax.axis_index("s")
        base = s * CPT2
        pltpu.sync_copy(e2_h.at[1, pl.ds(base, CPT2)], dst_v)
        pltpu.sync_copy(e2_h.at[0, pl.ds(base, BLK_I)], srcb_v.at[0])
        pltpu.async_copy(e2_h.at[0, pl.ds(base + BLK_I, BLK_I)],
                         srcb_v.at[1], sem_i)
        pltpu.sync_copy(zeros_h, acc_sh.at[pl.ds(s * ZR, ZR)])
        pltpu.sync_copy(y_h.at[c, pl.ds(s * ZR, ZR)],
                        y_t.at[pl.ds(s * ZR, ZR)])
        plsc.subcore_barrier()
        pltpu.async_copy(y_t.at[srcb_v.at[0, 0]], rows_v.at[0], sem_g)

        def outer(g, carry):
            gm = g % 2
            gn = (g + 1) % 2
            for k in range(BLK_I):
                j = g * BLK_I + k
                slot = rows_v.at[k % 2]
                nslot = rows_v.at[(k + 1) % 2]
                if k < BLK_I - 1:
                    pltpu.async_copy(y_t.at[srcb_v.at[gm, k + 1]], nslot,
                                     sem_g)
                else:
                    @pl.when(g + 1 < NBLK2)
                    def _():
                        pltpu.make_async_copy(
                            e2_h.at[0, pl.ds(base, BLK_I)],
                            srcb_v.at[gn], sem_i).wait()
                        pltpu.async_copy(y_t.at[srcb_v.at[gn, 0]], nslot,
                                         sem_g)

                    @pl.when(g + 2 < NBLK2)
                    def _():
                        pltpu.async_copy(
                            e2_h.at[0, pl.ds(base + (g + 2) * BLK_I, BLK_I)],
                            srcb_v.at[gm], sem_i)

                pltpu.make_async_copy(y_t.at[srcb_v.at[gm, k]], slot,
                                      sem_g).wait()
                pltpu.sync_copy(slot, acc_sh.at[dst_v.at[j]], add=True)
            return carry

        lax.fori_loop(0, NBLK2, outer, 0, unroll=False)
        plsc.subcore_barrier()
        pltpu.sync_copy(acc_sh.at[pl.ds(s * ZR, ZR)],
                        out_h.at[c, pl.ds(s * ZR, ZR)])

    return scat_kernel(y2h, e2, zeros64)


def _norms(dref):
    d = dref[0] + dref[1]
    n_src = lax.rsqrt(jnp.maximum(d[:, 0:1], 1.0))
    n_dst = lax.rsqrt(jnp.maximum(d[:, 1:2], 1.0))
    return n_src, n_dst


_BLK = 1024


def _tc_prep(x_pad, W1, deg_p):
    def body(x_ref, w_ref, d_ref, o_ref):
        n_src, _ = _norms(d_ref)
        y = jnp.dot(x_ref[...], w_ref[...],
                    preferred_element_type=jnp.float32) * n_src
        o_ref[0] = y[:, :DHH].astype(jnp.bfloat16)
        o_ref[1] = y[:, DHH:].astype(jnp.bfloat16)

    return pl.pallas_call(
        body,
        grid=(NPAD // _BLK,),
        in_specs=[
            pl.BlockSpec((_BLK, DIN), lambda i: (i, 0)),
            pl.BlockSpec((DIN, DH), lambda i: (0, 0)),
            pl.BlockSpec((NC, _BLK, DW), lambda i: (0, i, 0)),
        ],
        out_specs=pl.BlockSpec((NC, _BLK, DHH), lambda i: (0, i, 0)),
        out_shape=jax.ShapeDtypeStruct((NC, NPAD, DHH), jnp.bfloat16),
    )(x_pad, W1, deg_p)


def _tc_mid(agg1, deg_p, W2, b1r):
    def body(a_ref, d_ref, w_ref, bias_ref, o_ref):
        n_src, n_dst = _norms(d_ref)
        h = jnp.concatenate([a_ref[0], a_ref[1]],
                            axis=1).astype(jnp.float32) * n_dst + bias_ref[...]
        h = jnp.maximum(h, 0.0)
        o_ref[...] = jnp.dot(h, w_ref[...],
                             preferred_element_type=jnp.float32) * n_src

    return pl.pallas_call(
        body,
        grid=(NPAD // _BLK,),
        in_specs=[
            pl.BlockSpec((NC, _BLK, DHH), lambda i: (0, i, 0)),
            pl.BlockSpec((NC, _BLK, DW), lambda i: (0, i, 0)),
            pl.BlockSpec((DH, DOUT), lambda i: (0, 0)),
            pl.BlockSpec((1, DH), lambda i: (0, 0)),
        ],
        out_specs=pl.BlockSpec((_BLK, DOUT), lambda i: (i, 0)),
        out_shape=jax.ShapeDtypeStruct((NPAD, DOUT), jnp.float32),
    )(agg1, deg_p, W2, b1r)


def _tc_final(agg2, deg_p, b2r):
    def body(a_ref, d_ref, b_ref, o_ref):
        _, n_dst = _norms(d_ref)
        o_ref[...] = (a_ref[0] + a_ref[1]) * n_dst + b_ref[...]

    return pl.pallas_call(
        body,
        grid=(NPAD // _BLK,),
        in_specs=[
            pl.BlockSpec((NC, _BLK, DOUT), lambda i: (0, i, 0)),
            pl.BlockSpec((NC, _BLK, DW), lambda i: (0, i, 0)),
            pl.BlockSpec((1, DOUT), lambda i: (0, 0)),
        ],
        out_specs=pl.BlockSpec((_BLK, DOUT), lambda i: (i, 0)),
        out_shape=jax.ShapeDtypeStruct((NPAD, DOUT), jnp.float32),
    )(agg2, deg_p, b2r)


def kernel(in_feat, edge_index, W1, b1, W2, b2):
    pad_e = EPAD - N_EDGES_
    epad = jnp.full((2, pad_e), DUMMY, dtype=jnp.int32)
    e2 = jnp.concatenate([edge_index, epad], axis=1).reshape(2, NCHUNKS_PAD, CHUNK)
    x_pad = jnp.zeros((NPAD, DIN), jnp.float32).at[:N_NODES_].set(in_feat)
    b1r = b1.reshape(1, DH)
    b2r = b2.reshape(1, DOUT)
    eye2 = jnp.zeros((2, DW), jnp.float32).at[0, 0].set(1.0).at[1, 1].set(1.0)
    cvals = jnp.tile(eye2[:, None, :], (1, CHUNK, 1))
    zeros2 = jnp.zeros((ZR, DW), jnp.float32)
    zeros64 = jnp.zeros((ZR, DHH), jnp.bfloat16)
    zeros40 = jnp.zeros((ZR, DOUT), jnp.float32)

    deg_p = _sc_degree(e2, cvals, zeros2)
    y1 = _tc_prep(x_pad, W1, deg_p)
    agg1 = _sc_scatter_split(y1, e2, zeros64)
    y2 = _tc_mid(agg1, deg_p, W2, b1r)
    agg2 = _sc_scatter_40(y2, e2, zeros40)
    out = _tc_final(agg2, deg_p, b2r)
    return out[:N_NODES_]


# speedup vs baseline: 1.3382x; 1.0819x over previous
import functools

import jax
import jax.numpy as jnp
from jax import lax
from jax.experimental import pallas as pl
from jax.experimental.pallas import tpu as pltpu
from jax.experimental.pallas import tpu_sc as plsc

N_NODES_ = 10000
N_EDGES_ = 320000
DIN = 128
DH = 128
DOUT = 40

NC = 2
NS = 16
NW = NC * NS
CHUNK = 128
NPAD = 10240
DUMMY = N_NODES_
NCHUNKS = (N_EDGES_ + CHUNK - 1) // CHUNK
CPT = ((NCHUNKS + NW - 1) // NW + 7) // 8 * 8
NCHUNKS_PAD = CPT * NW
EPAD = NCHUNKS_PAD * CHUNK
ZR = NPAD // NS

_MESH = plsc.VectorSubcoreMesh(core_axis_name="c", subcore_axis_name="s")


DW = 8


def _sc_degree(e2, cvals, zeros2):
    @functools.partial(
        pl.kernel,
        out_type=jax.ShapeDtypeStruct((NC, NPAD, DW), jnp.float32),
        mesh=_MESH,
        compiler_params=pltpu.CompilerParams(use_tc_tiling_on_sc=False),
        scratch_types=[
            pltpu.VMEM((CPT, CHUNK), jnp.int32),
            pltpu.VMEM((CPT, CHUNK), jnp.int32),
            pltpu.VMEM((CHUNK, DW), jnp.float32),
            pltpu.VMEM((CHUNK, DW), jnp.float32),
            pltpu.VMEM_SHARED((NPAD, DW), jnp.float32),
            pltpu.SemaphoreType.DMA,
        ],
    )
    def deg_kernel(e2_h, cvals_h, zeros_h, out_h, src_v, dst_v, va, vb,
                   acc_sh, sem_s):
        c = lax.axis_index("c")
        s = lax.axis_index("s")
        wid = s * NC + c
        pltpu.sync_copy(e2_h.at[0, pl.ds(wid * CPT, CPT)], src_v)
        pltpu.sync_copy(e2_h.at[1, pl.ds(wid * CPT, CPT)], dst_v)
        pltpu.sync_copy(cvals_h.at[0], va)
        pltpu.sync_copy(cvals_h.at[1], vb)
        pltpu.sync_copy(zeros_h, acc_sh.at[pl.ds(s * ZR, ZR)])
        plsc.subcore_barrier()

        def step(j, carry):
            pltpu.async_copy(va, acc_sh.at[src_v.at[j]], sem_s, add=True)
            pltpu.async_copy(vb, acc_sh.at[dst_v.at[j]], sem_s, add=True)
            return carry

        lax.fori_loop(0, CPT, step, 0, unroll=False)

        def drain(j, carry):
            pltpu.make_async_copy(va, acc_sh.at[src_v.at[0]], sem_s).wait()
            pltpu.make_async_copy(vb, acc_sh.at[dst_v.at[0]], sem_s).wait()
            return carry

        lax.fori_loop(0, CPT, drain, 0, unroll=False)
        plsc.subcore_barrier()
        pltpu.sync_copy(acc_sh.at[pl.ds(s * ZR, ZR)],
                        out_h.at[c, pl.ds(s * ZR, ZR)])

    return deg_kernel(e2, cvals, zeros2)


BLK_I = 8
NBLK = CPT // BLK_I


def _make_sc_scatter(d_feat, stage=False, dtype=jnp.float32):
    scratch = [
        pltpu.VMEM((2, BLK_I, CHUNK), jnp.int32),
        pltpu.VMEM((CPT, CHUNK), jnp.int32),
        pltpu.VMEM((2, CHUNK, d_feat), dtype),
        pltpu.VMEM_SHARED((NPAD, d_feat), dtype),
        pltpu.SemaphoreType.DMA,
        pltpu.SemaphoreType.DMA,
    ]
    if stage:
        scratch.append(pltpu.VMEM_SHARED((NPAD, d_feat), dtype))

    @functools.partial(
        pl.kernel,
        out_type=jax.ShapeDtypeStruct((NC, NPAD, d_feat), dtype),
        mesh=_MESH,
        compiler_params=pltpu.CompilerParams(use_tc_tiling_on_sc=False),
        scratch_types=scratch,
    )
    def scat_kernel(y_h, e2_h, zeros_h, out_h, srcb_v, dst_v, rows_v, acc_sh,
                    sem_g, sem_i, *maybe_ysh):
        c = lax.axis_index("c")
        s = lax.axis_index("s")
        wid = s * NC + c
        base = wid * CPT
        pltpu.sync_copy(e2_h.at[1, pl.ds(base, CPT)], dst_v)
        pltpu.sync_copy(e2_h.at[0, pl.ds(base, BLK_I)], srcb_v.at[0])
        pltpu.async_copy(e2_h.at[0, pl.ds(base + BLK_I, BLK_I)],
                         srcb_v.at[1], sem_i)
        pltpu.sync_copy(zeros_h, acc_sh.at[pl.ds(s * ZR, ZR)])
        if stage:
            y_t = maybe_ysh[0]
            pltpu.sync_copy(y_h.at[pl.ds(s * ZR, ZR)],
                            y_t.at[pl.ds(s * ZR, ZR)])
        else:
            y_t = y_h
        plsc.subcore_barrier()
        pltpu.async_copy(y_t.at[srcb_v.at[0, 0]], rows_v.at[0], sem_g)

        def outer(g, carry):
            gm = g % 2
            gn = (g + 1) % 2
            for k in range(BLK_I):
                j = g * BLK_I + k
                slot = rows_v.at[k % 2]
                nslot = rows_v.at[(k + 1) % 2]
                if k < BLK_I - 1:
                    pltpu.async_copy(y_t.at[srcb_v.at[gm, k + 1]], nslot,
                                     sem_g)
                else:
                    @pl.when(g + 1 < NBLK)
                    def _():
                        pltpu.make_async_copy(
                            e2_h.at[0, pl.ds(base, BLK_I)],
                            srcb_v.at[gn], sem_i).wait()
                        pltpu.async_copy(y_t.at[srcb_v.at[gn, 0]], nslot,
                                         sem_g)

                    @pl.when(g + 2 < NBLK)
                    def _():
                        pltpu.async_copy(
                            e2_h.at[0, pl.ds(base + (g + 2) * BLK_I, BLK_I)],
                            srcb_v.at[gm], sem_i)

                pltpu.make_async_copy(y_t.at[srcb_v.at[gm, k]], slot,
                                      sem_g).wait()
                pltpu.sync_copy(slot, acc_sh.at[dst_v.at[j]], add=True)
            return carry

        lax.fori_loop(0, NBLK, outer, 0, unroll=False)
        plsc.subcore_barrier()
        pltpu.sync_copy(acc_sh.at[pl.ds(s * ZR, ZR)],
                        out_h.at[c, pl.ds(s * ZR, ZR)])

    return scat_kernel


DHH = DH // 2
DOUT_P = 48
_sc_scatter_48 = _make_sc_scatter(DOUT_P, stage=True, dtype=jnp.bfloat16)

CPT2 = NCHUNKS_PAD // NS
NBLK2 = CPT2 // BLK_I


def _sc_scatter_split(y2h, e2, zeros64):
    @functools.partial(
        pl.kernel,
        out_type=jax.ShapeDtypeStruct((NC, NPAD, DHH), jnp.bfloat16),
        mesh=_MESH,
        compiler_params=pltpu.CompilerParams(use_tc_tiling_on_sc=False),
        scratch_types=[
            pltpu.VMEM((2, BLK_I, CHUNK), jnp.int32),
            pltpu.VMEM((CPT2, CHUNK), jnp.int32),
            pltpu.VMEM((2, CHUNK, DHH), jnp.bfloat16),
            pltpu.VMEM_SHARED((NPAD, DHH), jnp.bfloat16),
            pltpu.VMEM_SHARED((NPAD, DHH), jnp.bfloat16),
            pltpu.SemaphoreType.DMA,
            pltpu.SemaphoreType.DMA,
        ],
    )
    def scat_kernel(y_h, e2_h, zeros_h, out_h, srcb_v, dst_v, rows_v, acc_sh,
                    y_t, sem_g, sem_i):
        c = lax.axis_index("c")
        s = lax.axis_index("s")
        base = s * CPT2
        pltpu.sync_copy(e2_h.at[1, pl.ds(base, CPT2)], dst_v)
        pltpu.sync_copy(e2_h.at[0, pl.ds(base, BLK_I)], srcb_v.at[0])
        pltpu.async_copy(e2_h.at[0, pl.ds(base + BLK_I, BLK_I)],
                         srcb_v.at[1], sem_i)
        pltpu.sync_copy(zeros_h, acc_sh.at[pl.ds(s * ZR, ZR)])
        pltpu.sync_copy(y_h.at[c, pl.ds(s * ZR, ZR)],
                        y_t.at[pl.ds(s * ZR, ZR)])
        plsc.subcore_barrier()
        pltpu.async_copy(y_t.at[srcb_v.at[0, 0]], rows_v.at[0], sem_g)

        def outer(g, carry):
            gm = g % 2
            gn = (g + 1) % 2
            for k in range(BLK_I):
                j = g * BLK_I + k
                slot = rows_v.at[k % 2]
                nslot = rows_v.at[(k + 1) % 2]
                if k < BLK_I - 1:
                    pltpu.async_copy(y_t.at[srcb_v.at[gm, k + 1]], nslot,
                                     sem_g)
                else:
                    @pl.when(g + 1 < NBLK2)
                    def _():
                        pltpu.make_async_copy(
                            e2_h.at[0, pl.ds(base, BLK_I)],
                            srcb_v.at[gn], sem_i).wait()
                        pltpu.async_copy(y_t.at[srcb_v.at[gn, 0]], nslot,
                                         sem_g)

                    @pl.when(g + 2 < NBLK2)
                    def _():
                        pltpu.async_copy(
                            e2_h.at[0, pl.ds(base + (g + 2) * BLK_I, BLK_I)],
                            srcb_v.at[gm], sem_i)

                pltpu.make_async_copy(y_t.at[srcb_v.at[gm, k]], slot,
                                      sem_g).wait()
                pltpu.sync_copy(slot, acc_sh.at[dst_v.at[j]], add=True)
            return carry

        lax.fori_loop(0, NBLK2, outer, 0, unroll=False)
        plsc.subcore_barrier()
        pltpu.sync_copy(acc_sh.at[pl.ds(s * ZR, ZR)],
                        out_h.at[c, pl.ds(s * ZR, ZR)])

    return scat_kernel(y2h, e2, zeros64)


def _norms(dref):
    d = dref[0] + dref[1]
    n_src = lax.rsqrt(jnp.maximum(d[:, 0:1], 1.0))
    n_dst = lax.rsqrt(jnp.maximum(d[:, 1:2], 1.0))
    return n_src, n_dst


_BLK = 1024


def _tc_prep(x_pad, W1, deg_p):
    def body(x_ref, w_ref, d_ref, o_ref):
        n_src, _ = _norms(d_ref)
        y = jnp.dot(x_ref[...], w_ref[...],
                    preferred_element_type=jnp.float32) * n_src
        o_ref[0] = y[:, :DHH].astype(jnp.bfloat16)
        o_ref[1] = y[:, DHH:].astype(jnp.bfloat16)

    return pl.pallas_call(
        body,
        grid=(NPAD // _BLK,),
        in_specs=[
            pl.BlockSpec((_BLK, DIN), lambda i: (i, 0)),
            pl.BlockSpec((DIN, DH), lambda i: (0, 0)),
            pl.BlockSpec((NC, _BLK, DW), lambda i: (0, i, 0)),
        ],
        out_specs=pl.BlockSpec((NC, _BLK, DHH), lambda i: (0, i, 0)),
        out_shape=jax.ShapeDtypeStruct((NC, NPAD, DHH), jnp.bfloat16),
    )(x_pad, W1, deg_p)


def _tc_mid(agg1, deg_p, W2p, b1r):
    def body(a_ref, d_ref, w_ref, bias_ref, o_ref):
        n_src, n_dst = _norms(d_ref)
        h = jnp.concatenate([a_ref[0], a_ref[1]],
                            axis=1).astype(jnp.float32) * n_dst + bias_ref[...]
        h = jnp.maximum(h, 0.0)
        o_ref[...] = (jnp.dot(h, w_ref[...],
                              preferred_element_type=jnp.float32)
                      * n_src).astype(jnp.bfloat16)

    return pl.pallas_call(
        body,
        grid=(NPAD // _BLK,),
        in_specs=[
            pl.BlockSpec((NC, _BLK, DHH), lambda i: (0, i, 0)),
            pl.BlockSpec((NC, _BLK, DW), lambda i: (0, i, 0)),
            pl.BlockSpec((DH, DOUT_P), lambda i: (0, 0)),
            pl.BlockSpec((1, DH), lambda i: (0, 0)),
        ],
        out_specs=pl.BlockSpec((_BLK, DOUT_P), lambda i: (i, 0)),
        out_shape=jax.ShapeDtypeStruct((NPAD, DOUT_P), jnp.bfloat16),
    )(agg1, deg_p, W2p, b1r)


def _tc_final(agg2, deg_p, b2r):
    def body(a_ref, d_ref, b_ref, o_ref):
        _, n_dst = _norms(d_ref)
        agg = (a_ref[0].astype(jnp.float32) + a_ref[1].astype(jnp.float32))
        o_ref[...] = agg[:, :DOUT] * n_dst + b_ref[...]

    return pl.pallas_call(
        body,
        grid=(NPAD // _BLK,),
        in_specs=[
            pl.BlockSpec((NC, _BLK, DOUT_P), lambda i: (0, i, 0)),
            pl.BlockSpec((NC, _BLK, DW), lambda i: (0, i, 0)),
            pl.BlockSpec((1, DOUT), lambda i: (0, 0)),
        ],
        out_specs=pl.BlockSpec((_BLK, DOUT), lambda i: (i, 0)),
        out_shape=jax.ShapeDtypeStruct((NPAD, DOUT), jnp.float32),
    )(agg2, deg_p, b2r)


def kernel(in_feat, edge_index, W1, b1, W2, b2):
    pad_e = EPAD - N_EDGES_
    epad = jnp.full((2, pad_e), DUMMY, dtype=jnp.int32)
    e2 = jnp.concatenate([edge_index, epad], axis=1).reshape(2, NCHUNKS_PAD, CHUNK)
    x_pad = jnp.zeros((NPAD, DIN), jnp.float32).at[:N_NODES_].set(in_feat)
    b1r = b1.reshape(1, DH)
    b2r = b2.reshape(1, DOUT)
    eye2 = jnp.zeros((2, DW), jnp.float32).at[0, 0].set(1.0).at[1, 1].set(1.0)
    cvals = jnp.tile(eye2[:, None, :], (1, CHUNK, 1))
    zeros2 = jnp.zeros((ZR, DW), jnp.float32)
    zeros64 = jnp.zeros((ZR, DHH), jnp.bfloat16)
    zeros48 = jnp.zeros((ZR, DOUT_P), jnp.bfloat16)
    W2p = jnp.zeros((DH, DOUT_P), jnp.float32).at[:, :DOUT].set(W2)

    deg_p = _sc_degree(e2, cvals, zeros2)
    y1 = _tc_prep(x_pad, W1, deg_p)
    agg1 = _sc_scatter_split(y1, e2, zeros64)
    y2 = _tc_mid(agg1, deg_p, W2p, b1r)
    agg2 = _sc_scatter_48(y2, e2, zeros48)
    out = _tc_final(agg2, deg_p, b2r)
    return out[:N_NODES_]


# speedup vs baseline: 1.4064x; 1.0509x over previous
import functools

import jax
import jax.numpy as jnp
from jax import lax
from jax.experimental import pallas as pl
from jax.experimental.pallas import tpu as pltpu
from jax.experimental.pallas import tpu_sc as plsc

N_NODES_ = 10000
N_EDGES_ = 320000
DIN = 128
DH = 128
DOUT = 40

NC = 2
NS = 16
NW = NC * NS
CHUNK = 128
NPAD = 10240
DUMMY = N_NODES_
NCHUNKS = (N_EDGES_ + CHUNK - 1) // CHUNK
CPT = ((NCHUNKS + NW - 1) // NW + 7) // 8 * 8
NCHUNKS_PAD = CPT * NW
EPAD = NCHUNKS_PAD * CHUNK
ZR = NPAD // NS

_MESH = plsc.VectorSubcoreMesh(core_axis_name="c", subcore_axis_name="s")


DW = 8


def _sc_degree(e2, cvals, zeros2):
    @functools.partial(
        pl.kernel,
        out_type=jax.ShapeDtypeStruct((NC, NPAD, DW), jnp.float32),
        mesh=_MESH,
        compiler_params=pltpu.CompilerParams(use_tc_tiling_on_sc=False),
        scratch_types=[
            pltpu.VMEM((CPT, CHUNK), jnp.int32),
            pltpu.VMEM((CPT, CHUNK), jnp.int32),
            pltpu.VMEM((CHUNK, DW), jnp.float32),
            pltpu.VMEM((CHUNK, DW), jnp.float32),
            pltpu.VMEM_SHARED((NPAD, DW), jnp.float32),
            pltpu.SemaphoreType.DMA,
        ],
    )
    def deg_kernel(e2_h, cvals_h, zeros_h, out_h, src_v, dst_v, va, vb,
                   acc_sh, sem_s):
        c = lax.axis_index("c")
        s = lax.axis_index("s")
        wid = s * NC + c
        pltpu.sync_copy(e2_h.at[0, pl.ds(wid * CPT, CPT)], src_v)
        pltpu.sync_copy(e2_h.at[1, pl.ds(wid * CPT, CPT)], dst_v)
        pltpu.sync_copy(cvals_h.at[0], va)
        pltpu.sync_copy(cvals_h.at[1], vb)
        pltpu.sync_copy(zeros_h, acc_sh.at[pl.ds(s * ZR, ZR)])
        plsc.subcore_barrier()

        def step(j, carry):
            pltpu.async_copy(va, acc_sh.at[src_v.at[j]], sem_s, add=True)
            pltpu.async_copy(vb, acc_sh.at[dst_v.at[j]], sem_s, add=True)
            return carry

        lax.fori_loop(0, CPT, step, 0, unroll=False)

        def drain(j, carry):
            pltpu.make_async_copy(va, acc_sh.at[src_v.at[0]], sem_s).wait()
            pltpu.make_async_copy(vb, acc_sh.at[dst_v.at[0]], sem_s).wait()
            return carry

        lax.fori_loop(0, CPT, drain, 0, unroll=False)
        plsc.subcore_barrier()
        pltpu.sync_copy(acc_sh.at[pl.ds(s * ZR, ZR)],
                        out_h.at[c, pl.ds(s * ZR, ZR)])

    return deg_kernel(e2, cvals, zeros2)


BLK_I = 8
NBLK = CPT // BLK_I

NR = 4


def _edge_loop(y_t, e2_h, base, nblk, srcb_v, dst_v, rows_v, acc_sh,
               sem_g, sem_s, sem_i):
    def wait_scatter():
        pltpu.make_async_copy(rows_v.at[0], acc_sh.at[dst_v.at[0]],
                              sem_s).wait()

    pltpu.async_copy(y_t.at[srcb_v.at[0, 0]], rows_v.at[0], sem_g)
    pltpu.async_copy(y_t.at[srcb_v.at[0, 1]], rows_v.at[1], sem_g)

    def outer(g, carry):
        gm = g % 2
        gn = (g + 1) % 2
        for k in range(BLK_I):
            j = g * BLK_I + k
            slot = rows_v.at[k % NR]
            pltpu.make_async_copy(y_t.at[srcb_v.at[gm, k]], slot,
                                  sem_g).wait()
            pltpu.async_copy(slot, acc_sh.at[dst_v.at[j]], sem_s, add=True)
            nslot = rows_v.at[(k + 2) % NR]
            if k < BLK_I - 2:
                if k < 2:
                    @pl.when(g > 0)
                    def _():
                        wait_scatter()
                else:
                    wait_scatter()
                pltpu.async_copy(y_t.at[srcb_v.at[gm, k + 2]], nslot, sem_g)
            elif k == BLK_I - 2:
                @pl.when(g + 1 < nblk)
                def _():
                    pltpu.make_async_copy(e2_h.at[0, pl.ds(base, BLK_I)],
                                          srcb_v.at[gn], sem_i).wait()
                    wait_scatter()
                    pltpu.async_copy(y_t.at[srcb_v.at[gn, 0]], nslot, sem_g)
            else:
                @pl.when(g + 1 < nblk)
                def _():
                    wait_scatter()
                    pltpu.async_copy(y_t.at[srcb_v.at[gn, 1]], nslot, sem_g)

                @pl.when(g + 2 < nblk)
                def _():
                    pltpu.async_copy(
                        e2_h.at[0, pl.ds(base + (g + 2) * BLK_I, BLK_I)],
                        srcb_v.at[gm], sem_i)
        return carry

    lax.fori_loop(0, nblk, outer, 0, unroll=False)
    for _ in range(NR):
        wait_scatter()


def _make_sc_scatter(d_feat, stage=False, dtype=jnp.float32):
    scratch = [
        pltpu.VMEM((2, BLK_I, CHUNK), jnp.int32),
        pltpu.VMEM((CPT, CHUNK), jnp.int32),
        pltpu.VMEM((NR, CHUNK, d_feat), dtype),
        pltpu.VMEM_SHARED((NPAD, d_feat), dtype),
        pltpu.SemaphoreType.DMA,
        pltpu.SemaphoreType.DMA,
        pltpu.SemaphoreType.DMA,
    ]
    if stage:
        scratch.append(pltpu.VMEM_SHARED((NPAD, d_feat), dtype))

    @functools.partial(
        pl.kernel,
        out_type=jax.ShapeDtypeStruct((NC, NPAD, d_feat), dtype),
        mesh=_MESH,
        compiler_params=pltpu.CompilerParams(use_tc_tiling_on_sc=False),
        scratch_types=scratch,
    )
    def scat_kernel(y_h, e2_h, zeros_h, out_h, srcb_v, dst_v, rows_v, acc_sh,
                    sem_g, sem_s, sem_i, *maybe_ysh):
        c = lax.axis_index("c")
        s = lax.axis_index("s")
        wid = s * NC + c
        base = wid * CPT
        pltpu.sync_copy(e2_h.at[1, pl.ds(base, CPT)], dst_v)
        pltpu.sync_copy(e2_h.at[0, pl.ds(base, BLK_I)], srcb_v.at[0])
        pltpu.async_copy(e2_h.at[0, pl.ds(base + BLK_I, BLK_I)],
                         srcb_v.at[1], sem_i)
        pltpu.sync_copy(zeros_h, acc_sh.at[pl.ds(s * ZR, ZR)])
        if stage:
            y_t = maybe_ysh[0]
            pltpu.sync_copy(y_h.at[pl.ds(s * ZR, ZR)],
                            y_t.at[pl.ds(s * ZR, ZR)])
        else:
            y_t = y_h
        plsc.subcore_barrier()
        _edge_loop(y_t, e2_h, base, NBLK, srcb_v, dst_v, rows_v, acc_sh,
                   sem_g, sem_s, sem_i)
        plsc.subcore_barrier()
        pltpu.sync_copy(acc_sh.at[pl.ds(s * ZR, ZR)],
                        out_h.at[c, pl.ds(s * ZR, ZR)])

    return scat_kernel


DHH = DH // 2
DOUT_P = 48
_sc_scatter_48 = _make_sc_scatter(DOUT_P, stage=True, dtype=jnp.bfloat16)

CPT2 = NCHUNKS_PAD // NS
NBLK2 = CPT2 // BLK_I


def _sc_scatter_split(y2h, e2, zeros64):
    @functools.partial(
        pl.kernel,
        out_type=jax.ShapeDtypeStruct((NC, NPAD, DHH), jnp.bfloat16),
        mesh=_MESH,
        compiler_params=pltpu.CompilerParams(use_tc_tiling_on_sc=False),
        scratch_types=[
            pltpu.VMEM((2, BLK_I, CHUNK), jnp.int32),
            pltpu.VMEM((CPT2, CHUNK), jnp.int32),
            pltpu.VMEM((NR, CHUNK, DHH), jnp.bfloat16),
            pltpu.VMEM_SHARED((NPAD, DHH), jnp.bfloat16),
            pltpu.VMEM_SHARED((NPAD, DHH), jnp.bfloat16),
            pltpu.SemaphoreType.DMA,
            pltpu.SemaphoreType.DMA,
            pltpu.SemaphoreType.DMA,
        ],
    )
    def scat_kernel(y_h, e2_h, zeros_h, out_h, srcb_v, dst_v, rows_v, acc_sh,
                    y_t, sem_g, sem_s, sem_i):
        c = lax.axis_index("c")
        s = lax.axis_index("s")
        base = s * CPT2
        pltpu.sync_copy(e2_h.at[1, pl.ds(base, CPT2)], dst_v)
        pltpu.sync_copy(e2_h.at[0, pl.ds(base, BLK_I)], srcb_v.at[0])
        pltpu.async_copy(e2_h.at[0, pl.ds(base + BLK_I, BLK_I)],
                         srcb_v.at[1], sem_i)
        pltpu.sync_copy(zeros_h, acc_sh.at[pl.ds(s * ZR, ZR)])
        pltpu.sync_copy(y_h.at[c, pl.ds(s * ZR, ZR)],
                        y_t.at[pl.ds(s * ZR, ZR)])
        plsc.subcore_barrier()
        _edge_loop(y_t, e2_h, base, NBLK2, srcb_v, dst_v, rows_v, acc_sh,
                   sem_g, sem_s, sem_i)
        plsc.subcore_barrier()
        pltpu.sync_copy(acc_sh.at[pl.ds(s * ZR, ZR)],
                        out_h.at[c, pl.ds(s * ZR, ZR)])

    return scat_kernel(y2h, e2, zeros64)


def _norms(dref):
    d = dref[0] + dref[1]
    n_src = lax.rsqrt(jnp.maximum(d[:, 0:1], 1.0))
    n_dst = lax.rsqrt(jnp.maximum(d[:, 1:2], 1.0))
    return n_src, n_dst


_BLK = 1024


def _tc_prep(x_pad, W1, deg_p):
    def body(x_ref, w_ref, d_ref, o_ref):
        n_src, _ = _norms(d_ref)
        y = jnp.dot(x_ref[...], w_ref[...],
                    preferred_element_type=jnp.float32) * n_src
        o_ref[0] = y[:, :DHH].astype(jnp.bfloat16)
        o_ref[1] = y[:, DHH:].astype(jnp.bfloat16)

    return pl.pallas_call(
        body,
        grid=(NPAD // _BLK,),
        in_specs=[
            pl.BlockSpec((_BLK, DIN), lambda i: (i, 0)),
            pl.BlockSpec((DIN, DH), lambda i: (0, 0)),
            pl.BlockSpec((NC, _BLK, DW), lambda i: (0, i, 0)),
        ],
        out_specs=pl.BlockSpec((NC, _BLK, DHH), lambda i: (0, i, 0)),
        out_shape=jax.ShapeDtypeStruct((NC, NPAD, DHH), jnp.bfloat16),
    )(x_pad, W1, deg_p)


def _tc_mid(agg1, deg_p, W2p, b1r):
    def body(a_ref, d_ref, w_ref, bias_ref, o_ref):
        n_src, n_dst = _norms(d_ref)
        h = jnp.concatenate([a_ref[0], a_ref[1]],
                            axis=1).astype(jnp.float32) * n_dst + bias_ref[...]
        h = jnp.maximum(h, 0.0)
        o_ref[...] = (jnp.dot(h, w_ref[...],
                              preferred_element_type=jnp.float32)
                      * n_src).astype(jnp.bfloat16)

    return pl.pallas_call(
        body,
        grid=(NPAD // _BLK,),
        in_specs=[
            pl.BlockSpec((NC, _BLK, DHH), lambda i: (0, i, 0)),
            pl.BlockSpec((NC, _BLK, DW), lambda i: (0, i, 0)),
            pl.BlockSpec((DH, DOUT_P), lambda i: (0, 0)),
            pl.BlockSpec((1, DH), lambda i: (0, 0)),
        ],
        out_specs=pl.BlockSpec((_BLK, DOUT_P), lambda i: (i, 0)),
        out_shape=jax.ShapeDtypeStruct((NPAD, DOUT_P), jnp.bfloat16),
    )(agg1, deg_p, W2p, b1r)


def _tc_final(agg2, deg_p, b2r):
    def body(a_ref, d_ref, b_ref, o_ref):
        _, n_dst = _norms(d_ref)
        agg = (a_ref[0].astype(jnp.float32) + a_ref[1].astype(jnp.float32))
        o_ref[...] = agg[:, :DOUT] * n_dst + b_ref[...]

    return pl.pallas_call(
        body,
        grid=(NPAD // _BLK,),
        in_specs=[
            pl.BlockSpec((NC, _BLK, DOUT_P), lambda i: (0, i, 0)),
            pl.BlockSpec((NC, _BLK, DW), lambda i: (0, i, 0)),
            pl.BlockSpec((1, DOUT), lambda i: (0, 0)),
        ],
        out_specs=pl.BlockSpec((_BLK, DOUT), lambda i: (i, 0)),
        out_shape=jax.ShapeDtypeStruct((NPAD, DOUT), jnp.float32),
    )(agg2, deg_p, b2r)


def kernel(in_feat, edge_index, W1, b1, W2, b2):
    pad_e = EPAD - N_EDGES_
    epad = jnp.full((2, pad_e), DUMMY, dtype=jnp.int32)
    e2 = jnp.concatenate([edge_index, epad], axis=1).reshape(2, NCHUNKS_PAD, CHUNK)
    x_pad = jnp.zeros((NPAD, DIN), jnp.float32).at[:N_NODES_].set(in_feat)
    b1r = b1.reshape(1, DH)
    b2r = b2.reshape(1, DOUT)
    eye2 = jnp.zeros((2, DW), jnp.float32).at[0, 0].set(1.0).at[1, 1].set(1.0)
    cvals = jnp.tile(eye2[:, None, :], (1, CHUNK, 1))
    zeros2 = jnp.zeros((ZR, DW), jnp.float32)
    zeros64 = jnp.zeros((ZR, DHH), jnp.bfloat16)
    zeros48 = jnp.zeros((ZR, DOUT_P), jnp.bfloat16)
    W2p = jnp.zeros((DH, DOUT_P), jnp.float32).at[:, :DOUT].set(W2)

    deg_p = _sc_degree(e2, cvals, zeros2)
    y1 = _tc_prep(x_pad, W1, deg_p)
    agg1 = _sc_scatter_split(y1, e2, zeros64)
    y2 = _tc_mid(agg1, deg_p, W2p, b1r)
    agg2 = _sc_scatter_48(y2, e2, zeros48)
    out = _tc_final(agg2, deg_p, b2r)
    return out[:N_NODES_]
